# baseline (speedup 1.0000x reference)
import functools
import math

import jax
import jax.numpy as jnp
from jax import lax
from jax.experimental import pallas as pl
from jax.experimental.pallas import tpu as pltpu
from jax.experimental.pallas import tpu_sc as plsc

_SCALE = 64.0
_MARGIN = 0.5
_COS_M = math.cos(_MARGIN)
_SIN_M = math.sin(_MARGIN)
_THETA = math.cos(math.pi - _MARGIN)
_SINMM = math.sin(math.pi - _MARGIN) * _MARGIN

_B = 1024
_V = 100000
_L = 16

_NC = 2
_NS = 16
_NW = _NC * _NS
_RPW = _B // _NW
_EPW = _RPW * _V

_CH = 32000
_NCH = _EPW // _CH


def _margin16(t):
    x = 1.0 - t * t
    i = lax.bitcast_convert_type(x, jnp.int32)
    i = jnp.int32(0x5F3759DF) - lax.shift_right_logical(i, 1)
    r = lax.bitcast_convert_type(i, jnp.float32)
    for _ in range(3):
        r = r * (1.5 - (0.5 * x) * r * r)
    sin_t = x * r
    ctm = t * _COS_M - sin_t * _SIN_M
    return jnp.where(t > _THETA, ctm, t - _SINMM)


def _sc_body(lg_hbm, fidx_hbm, vmask_hbm, out_hbm,
             fidx_v, vmask_v, tgt_v, res_v, ibuf, obuf, sem):
    wid = lax.axis_index("s") * _NC + lax.axis_index("c")
    base = wid * _RPW
    region = wid * _EPW
    pltpu.sync_copy(fidx_hbm.at[pl.ds(base, _RPW)], fidx_v)
    pltpu.sync_copy(vmask_hbm.at[pl.ds(base, _RPW)], vmask_v)

    @pl.loop(0, _NCH)
    def chunk_body(c):
        start = region + c * _CH
        pltpu.sync_copy(lg_hbm.at[pl.ds(start, _CH)], ibuf)

        @pl.loop(0, _CH // _L, unroll=8)
        def vec_body(j):
            obuf[pl.ds(j * _L, _L)] = ibuf[pl.ds(j * _L, _L)] * _SCALE

        pltpu.sync_copy(obuf, out_hbm.at[pl.ds(start, _CH)])

    pltpu.async_copy(lg_hbm.at[fidx_v], tgt_v, sem).wait()
    for h in range(_RPW // _L):
        t = tgt_v[pl.ds(h * _L, _L)]
        valid = vmask_v[pl.ds(h * _L, _L)] > 0
        res_v[pl.ds(h * _L, _L)] = jnp.where(valid, _margin16(t), t) * _SCALE
    pltpu.async_copy(res_v, out_hbm.at[fidx_v], sem).wait()


_sc_run = functools.partial(
    pl.kernel,
    mesh=plsc.VectorSubcoreMesh(core_axis_name="c", subcore_axis_name="s"),
    out_type=jax.ShapeDtypeStruct((_B * _V,), jnp.float32),
    scratch_types=[
        pltpu.VMEM((_RPW,), jnp.int32),
        pltpu.VMEM((_RPW,), jnp.int32),
        pltpu.VMEM((_RPW,), jnp.float32),
        pltpu.VMEM((_RPW,), jnp.float32),
        pltpu.VMEM((_CH,), jnp.float32),
        pltpu.VMEM((_CH,), jnp.float32),
        pltpu.SemaphoreType.DMA,
    ],
)


def kernel(logits, labels):
    rows = jnp.arange(_B, dtype=jnp.int32)
    valid = labels != -1
    safe = jnp.where(valid, labels, 0)
    fidx = rows * jnp.int32(_V) + safe
    vmask = valid.astype(jnp.int32)
    out_flat = _sc_run(_sc_body)(logits.reshape(_B * _V), fidx, vmask)
    return out_flat.reshape(_B, _V)


# speedup vs baseline: 1.4789x; 1.4789x over previous
import functools
import math

import jax
import jax.numpy as jnp
from jax import lax
from jax.experimental import pallas as pl
from jax.experimental.pallas import tpu as pltpu
from jax.experimental.pallas import tpu_sc as plsc

_SCALE = 64.0
_MARGIN = 0.5
_COS_M = math.cos(_MARGIN)
_SIN_M = math.sin(_MARGIN)
_THETA = math.cos(math.pi - _MARGIN)
_SINMM = math.sin(math.pi - _MARGIN) * _MARGIN

_B = 1024
_V = 100000
_L = 16

_NC = 2
_NS = 16
_NW = _NC * _NS
_RPW = _B // _NW
_EPW = _RPW * _V

_CH = 50000
_NCH = _EPW // _CH
_NVEC = _CH // _L


def _margin16(t):
    x = 1.0 - t * t
    i = lax.bitcast_convert_type(x, jnp.int32)
    i = jnp.int32(0x5F3759DF) - lax.shift_right_logical(i, 1)
    r = lax.bitcast_convert_type(i, jnp.float32)
    for _ in range(3):
        r = r * (1.5 - (0.5 * x) * r * r)
    sin_t = x * r
    ctm = t * _COS_M - sin_t * _SIN_M
    return jnp.where(t > _THETA, ctm, t - _SINMM)


def _sc_body(lg_hbm, fidx_hbm, vmask_hbm, out_hbm,
             fidx_v, vmask_v, tgt_v, res_v, buf0, buf1,
             isem0, isem1, osem0, osem1, sem):
    wid = lax.axis_index("s") * _NC + lax.axis_index("c")
    base = wid * _RPW
    region = wid * _EPW
    pltpu.sync_copy(fidx_hbm.at[pl.ds(base, _RPW)], fidx_v)
    pltpu.sync_copy(vmask_hbm.at[pl.ds(base, _RPW)], vmask_v)

    def in_cp(c, buf, sem_):
        return pltpu.make_async_copy(
            lg_hbm.at[pl.ds(region + c * _CH, _CH)], buf, sem_)

    def out_cp(c, buf, sem_):
        return pltpu.make_async_copy(
            buf, out_hbm.at[pl.ds(region + c * _CH, _CH)], sem_)

    def compute(buf):
        @plsc.parallel_loop(0, _NVEC, unroll=8)
        def vec_body(j):
            buf[pl.ds(j * _L, _L)] = buf[pl.ds(j * _L, _L)] * _SCALE

    in_cp(0, buf0, isem0).start()

    @pl.loop(0, _NCH // 2)
    def pipe_body(cc):
        c0 = cc * 2
        c1 = c0 + 1

        @pl.when(cc > 0)
        def _():
            out_cp(c1 - 2, buf1, osem1).wait()

        in_cp(c1, buf1, isem1).start()
        in_cp(c0, buf0, isem0).wait()
        compute(buf0)
        out_cp(c0, buf0, osem0).start()
        in_cp(c1, buf1, isem1).wait()
        compute(buf1)
        out_cp(c1, buf1, osem1).start()

        @pl.when(cc + 1 < _NCH // 2)
        def _():
            out_cp(c0, buf0, osem0).wait()
            in_cp(c0 + 2, buf0, isem0).start()

    out_cp(_NCH - 2, buf0, osem0).wait()
    out_cp(_NCH - 1, buf1, osem1).wait()

    pltpu.async_copy(lg_hbm.at[fidx_v], tgt_v, sem).wait()
    for h in range(_RPW // _L):
        t = tgt_v[pl.ds(h * _L, _L)]
        valid = vmask_v[pl.ds(h * _L, _L)] > 0
        res_v[pl.ds(h * _L, _L)] = jnp.where(valid, _margin16(t), t) * _SCALE
    pltpu.async_copy(res_v, out_hbm.at[fidx_v], sem).wait()


_sc_run = functools.partial(
    pl.kernel,
    mesh=plsc.VectorSubcoreMesh(core_axis_name="c", subcore_axis_name="s"),
    out_type=jax.ShapeDtypeStruct((_B * _V,), jnp.float32),
    scratch_types=[
        pltpu.VMEM((_RPW,), jnp.int32),
        pltpu.VMEM((_RPW,), jnp.int32),
        pltpu.VMEM((_RPW,), jnp.float32),
        pltpu.VMEM((_RPW,), jnp.float32),
        pltpu.VMEM((_CH,), jnp.float32),
        pltpu.VMEM((_CH,), jnp.float32),
        pltpu.SemaphoreType.DMA,
        pltpu.SemaphoreType.DMA,
        pltpu.SemaphoreType.DMA,
        pltpu.SemaphoreType.DMA,
        pltpu.SemaphoreType.DMA,
    ],
)


def kernel(logits, labels):
    rows = jnp.arange(_B, dtype=jnp.int32)
    valid = labels != -1
    safe = jnp.where(valid, labels, 0)
    fidx = rows * jnp.int32(_V) + safe
    vmask = valid.astype(jnp.int32)
    out_flat = _sc_run(_sc_body)(logits.reshape(_B * _V), fidx, vmask)
    return out_flat.reshape(_B, _V)
